# baseline (speedup 1.0000x reference)
import functools

import jax
import jax.numpy as jnp
from jax import lax
from jax.experimental import pallas as pl
from jax.experimental.pallas import tpu as pltpu
from jax.experimental.pallas import tpu_sc as plsc

_N = 10000
_E = 320000
_D = 128
_NC = 2
_NS = 16
_NW = _NC * _NS
_EPW = _E // _NW
_CHUNK = 80
_NCHUNK = _EPW // _CHUNK
_NBUF = 4
_GS = 8
_NGM = 15
_MAINC = _NGM * _GS
_TAIL = _NCHUNK - _MAINC
_NP = 10240
_RPS = _NP // _NS

_mesh = plsc.VectorSubcoreMesh(
    core_axis_name="c", subcore_axis_name="s", num_cores=_NC, num_subcores=_NS
)


@functools.partial(
    pl.kernel,
    out_type=jax.ShapeDtypeStruct((_NC, _NP, _D), jnp.float32),
    mesh=_mesh,
    scratch_types=(
        [
            pltpu.VMEM_SHARED((_NP, _D), jnp.float32),
            pltpu.VMEM((_GS, _CHUNK), jnp.int32),
            pltpu.VMEM((_GS, _CHUNK), jnp.int32),
            pltpu.VMEM((_GS, _CHUNK), jnp.float32),
            pltpu.VMEM((_GS, _CHUNK), jnp.int32),
            pltpu.VMEM((_GS, _CHUNK), jnp.int32),
            pltpu.VMEM((_GS, _CHUNK), jnp.float32),
        ]
        + [pltpu.VMEM((_CHUNK, _D), jnp.float32)] * _NBUF
        + [pltpu.SemaphoreType.DMA]
        + [pltpu.SemaphoreType.DMA] * (2 * _NBUF)
    ),
)
def _spmm_sc(col_hbm, row_hbm, val_hbm, emb_hbm, out_hbm,
             acc, curC, curR, curV, nxtC, nxtR, nxtV, *bufs_sems):
    rbuf = bufs_sems[:_NBUF]
    isem = bufs_sems[_NBUF]
    gsem = bufs_sems[_NBUF + 1:2 * _NBUF + 1]
    ssem = bufs_sems[2 * _NBUF + 1:]
    cid = lax.axis_index("c")
    sid = lax.axis_index("s")
    wid = sid * _NC + cid

    def zero_body(i, carry):
        for j in range(_D // 16):
            rbuf[0][i, pl.ds(j * 16, 16)] = jnp.zeros((16,), jnp.float32)
        return carry

    lax.fori_loop(0, _CHUNK, zero_body, 0)
    for t in range(_RPS // _CHUNK):
        pltpu.sync_copy(rbuf[0], acc.at[pl.ds(sid * _RPS + t * _CHUNK, _CHUNK)])
    plsc.subcore_barrier()

    def stage_group(g0, n):
        pltpu.async_copy(col_hbm.at[wid, pl.ds(g0, n)], nxtC.at[pl.ds(0, n)], isem)
        pltpu.async_copy(row_hbm.at[wid, pl.ds(g0, n)], nxtR.at[pl.ds(0, n)], isem)
        pltpu.async_copy(val_hbm.at[wid, pl.ds(g0, n)], nxtV.at[pl.ds(0, n)], isem)

    def wait_group(n):
        pltpu.make_async_copy(col_hbm.at[0, pl.ds(0, n)], nxtC.at[pl.ds(0, n)], isem).wait()
        pltpu.make_async_copy(row_hbm.at[0, pl.ds(0, n)], nxtR.at[pl.ds(0, n)], isem).wait()
        pltpu.make_async_copy(val_hbm.at[0, pl.ds(0, n)], nxtV.at[pl.ds(0, n)], isem).wait()

    def start_gather(idx_ref, b):
        pltpu.async_copy(emb_hbm.at[idx_ref], rbuf[b], gsem[b])

    def wait_gather(b):
        pltpu.make_async_copy(emb_hbm.at[curC.at[0]], rbuf[b], gsem[b]).wait()

    def start_scatter(idx_ref, b):
        pltpu.async_copy(rbuf[b], acc.at[idx_ref], ssem[b], add=True)

    def wait_scatter(b):
        pltpu.make_async_copy(rbuf[b], acc.at[curR.at[0]], ssem[b]).wait()

    def mul_rows(b, val_ref, j):
        def mul_body(g, c2):
            vblk = val_ref[j, pl.ds(g * 16, 16)]
            for e16 in range(16):
                s = vblk[e16]
                e = g * 16 + e16
                for d in range(_D // 16):
                    sl = pl.ds(d * 16, 16)
                    rbuf[b][e, sl] = rbuf[b][e, sl] * s
            return c2

        lax.fori_loop(0, _CHUNK // 16, mul_body, 0)

    stage_group(0, _GS)
    wait_group(_GS)
    start_gather(nxtC.at[0], 0)
    start_gather(nxtC.at[1], 1)

    def main_body(m, carry):
        i0 = m * _GS
        for j in range(_GS):
            i = i0 + j
            b = j % _NBUF

            @pl.when(i >= 1)
            def _wait_prev_scatter():
                wait_scatter((b + _NBUF - 1) % _NBUF)

            if j == 0:
                for src, dst in ((nxtC, curC), (nxtR, curR), (nxtV, curV)):
                    for r in range(_GS):
                        for g5 in range(_CHUNK // 16):
                            sl = pl.ds(g5 * 16, 16)
                            dst[r, sl] = src[r, sl]
            if j <= 5:
                start_gather(curC.at[j + 2], (b + 2) % _NBUF)

            wait_gather(b)

            if j == 1:
                @pl.when(m < _NGM - 1)
                def _stage_next():
                    stage_group(i0 + _GS, _GS)
            if j == 6 or j == 7:
                @pl.when(m < _NGM - 1)
                def _prefetch_next_group():
                    if j == 6:
                        wait_group(_GS)
                    start_gather(nxtC.at[j - 6], (b + 2) % _NBUF)

            mul_rows(b, curV, j)
            start_scatter(curR.at[j], b)
        return carry

    lax.fori_loop(0, _NGM, main_body, 0)

    stage_group(_MAINC, _TAIL)
    wait_group(_TAIL)
    start_gather(nxtC.at[0], 0)
    start_gather(nxtC.at[1], 1)
    for j in range(_TAIL):
        b = j % _NBUF
        wait_scatter((b + _NBUF - 1) % _NBUF)
        if j + 2 < _TAIL:
            start_gather(nxtC.at[j + 2], (b + 2) % _NBUF)
        wait_gather(b)
        mul_rows(b, nxtV, j)
        start_scatter(nxtR.at[j], b)
    wait_scatter((_TAIL - 1) % _NBUF)
    plsc.subcore_barrier()

    pltpu.sync_copy(
        acc.at[pl.ds(sid * _RPS, _RPS)],
        out_hbm.at[cid, pl.ds(sid * _RPS, _RPS)],
    )


def _combine_body(p_ref, o_ref):
    o_ref[...] = p_ref[0, :_N] + p_ref[1, :_N]


_combine = pl.pallas_call(
    _combine_body,
    out_shape=jax.ShapeDtypeStruct((_N, _D), jnp.float32),
)


@jax.jit
def kernel(adj_indices, adj_values, embeds):
    adj = adj_indices.astype(jnp.int32)
    col = adj[1].reshape(_NW, _NCHUNK, _CHUNK)
    row = adj[0].reshape(_NW, _NCHUNK, _CHUNK)
    val = adj_values.reshape(_NW, _NCHUNK, _CHUNK)
    partials = _spmm_sc(col, row, val, embeds)
    return _combine(partials)


# speedup vs baseline: 1.0127x; 1.0127x over previous
import functools

import jax
import jax.numpy as jnp
from jax import lax
from jax.experimental import pallas as pl
from jax.experimental.pallas import tpu as pltpu
from jax.experimental.pallas import tpu_sc as plsc

_N = 10000
_E = 320000
_D = 128
_NC = 2
_NS = 16
_NW = _NC * _NS
_EPW = _E // _NW
_CHUNK = 80
_NCHUNK = _EPW // _CHUNK
_NBUF = 4
_GS = 8
_GE = _GS * _CHUNK
_NGM = 15
_MAINC = _NGM * _GS
_TAIL = _NCHUNK - _MAINC
_NP = 10240
_RPS = _NP // _NS

_mesh = plsc.VectorSubcoreMesh(
    core_axis_name="c", subcore_axis_name="s", num_cores=_NC, num_subcores=_NS
)


@functools.partial(
    pl.kernel,
    out_type=jax.ShapeDtypeStruct((_NC, _NP, _D), jnp.float32),
    mesh=_mesh,
    scratch_types=(
        [
            pltpu.VMEM_SHARED((_NP, _D), jnp.float32),
            pltpu.VMEM((_GE,), jnp.int32),
            pltpu.VMEM((_GE,), jnp.float32),
            pltpu.VMEM((_GE,), jnp.int32),
            pltpu.VMEM((_GE,), jnp.float32),
        ]
        + [pltpu.VMEM((_CHUNK,), jnp.int32)] * _NBUF
        + [pltpu.VMEM((_CHUNK, _D), jnp.float32)] * _NBUF
        + [pltpu.SemaphoreType.DMA]
        + [pltpu.SemaphoreType.DMA] * (3 * _NBUF)
    ),
)
def _spmm_sc(col_hbm, row_hbm, val_hbm, emb_hbm, out_hbm,
             acc, curC, curV, nxtC, nxtV, *bufs_sems):
    rowb = bufs_sems[:_NBUF]
    rbuf = bufs_sems[_NBUF:2 * _NBUF]
    isem = bufs_sems[2 * _NBUF]
    rsem = bufs_sems[2 * _NBUF + 1:3 * _NBUF + 1]
    gsem = bufs_sems[3 * _NBUF + 1:4 * _NBUF + 1]
    ssem = bufs_sems[4 * _NBUF + 1:]
    cid = lax.axis_index("c")
    sid = lax.axis_index("s")
    wid = sid * _NC + cid
    e0 = wid * _EPW

    def zero_body(i, carry):
        for j in range(_D // 16):
            rbuf[0][i, pl.ds(j * 16, 16)] = jnp.zeros((16,), jnp.float32)
        return carry

    lax.fori_loop(0, _CHUNK, zero_body, 0)
    for t in range(_RPS // _CHUNK):
        pltpu.sync_copy(rbuf[0], acc.at[pl.ds(sid * _RPS + t * _CHUNK, _CHUNK)])
    plsc.subcore_barrier()

    def stage_group(g0, n):
        pltpu.async_copy(col_hbm.at[pl.ds(e0 + g0, n)], nxtC.at[pl.ds(0, n)], isem)
        pltpu.async_copy(val_hbm.at[pl.ds(e0 + g0, n)], nxtV.at[pl.ds(0, n)], isem)

    def wait_group(n):
        pltpu.make_async_copy(col_hbm.at[pl.ds(0, n)], nxtC.at[pl.ds(0, n)], isem).wait()
        pltpu.make_async_copy(val_hbm.at[pl.ds(0, n)], nxtV.at[pl.ds(0, n)], isem).wait()

    def stage_row(i, b):
        pltpu.async_copy(row_hbm.at[pl.ds(e0 + i * _CHUNK, _CHUNK)], rowb[b], rsem[b])

    def wait_row(b):
        pltpu.make_async_copy(row_hbm.at[pl.ds(0, _CHUNK)], rowb[b], rsem[b]).wait()

    def start_gather(col_ref, j, b):
        pltpu.async_copy(emb_hbm.at[col_ref.at[pl.ds(j * _CHUNK, _CHUNK)]],
                         rbuf[b], gsem[b])

    def wait_gather(b):
        pltpu.make_async_copy(emb_hbm.at[rowb[0]], rbuf[b], gsem[b]).wait()

    def start_scatter(b):
        pltpu.async_copy(rbuf[b], acc.at[rowb[b]], ssem[b], add=True)

    def wait_scatter(b):
        pltpu.make_async_copy(rbuf[b], acc.at[rowb[b]], ssem[b]).wait()

    def mul_rows(b, val_ref, j):
        def mul_body(g, c2):
            vblk = val_ref[pl.ds(j * _CHUNK + g * 16, 16)]
            for e16 in range(16):
                s = vblk[e16]
                e = g * 16 + e16
                for d in range(_D // 16):
                    sl = pl.ds(d * 16, 16)
                    rbuf[b][e, sl] = rbuf[b][e, sl] * s
            return c2

        lax.fori_loop(0, _CHUNK // 16, mul_body, 0)

    stage_group(0, _GE)
    stage_row(0, 0)
    stage_row(1, 1)
    stage_row(2, 2)
    wait_group(_GE)
    start_gather(nxtC, 0, 0)
    start_gather(nxtC, 1, 1)

    def main_body(m, carry):
        i0 = m * _GS
        for j in range(_GS):
            i = i0 + j
            b = j % _NBUF

            @pl.when(i >= 1)
            def _wait_prev_scatter():
                wait_scatter((b + _NBUF - 1) % _NBUF)

            stage_row(i + 3, (b + 3) % _NBUF)

            if j == 0:
                for src, dst in ((nxtC, curC), (nxtV, curV)):
                    for g5 in range(_GE // 16):
                        sl = pl.ds(g5 * 16, 16)
                        dst[sl] = src[sl]
            if j <= 5:
                start_gather(curC, j + 2, (b + 2) % _NBUF)

            wait_gather(b)

            if j == 1:
                @pl.when(m < _NGM - 1)
                def _stage_next():
                    stage_group(i0 * _CHUNK + _GE, _GE)
            if j == 6 or j == 7:
                @pl.when(m < _NGM - 1)
                def _prefetch_next_group():
                    if j == 6:
                        wait_group(_GE)
                    start_gather(nxtC, j - 6, (b + 2) % _NBUF)

            mul_rows(b, curV, j)
            wait_row(b)
            start_scatter(b)
        return carry

    lax.fori_loop(0, _NGM, main_body, 0)

    stage_group(_MAINC * _CHUNK, _TAIL * _CHUNK)
    wait_group(_TAIL * _CHUNK)
    start_gather(nxtC, 0, 0)
    start_gather(nxtC, 1, 1)
    for j in range(_TAIL):
        b = j % _NBUF
        wait_scatter((b + _NBUF - 1) % _NBUF)
        if _MAINC + j + 3 < _NCHUNK:
            stage_row(_MAINC + j + 3, (b + 3) % _NBUF)
        if j + 2 < _TAIL:
            start_gather(nxtC, j + 2, (b + 2) % _NBUF)
        wait_gather(b)
        mul_rows(b, nxtV, j)
        wait_row(b)
        start_scatter(b)
    wait_scatter((_TAIL - 1) % _NBUF)
    plsc.subcore_barrier()

    pltpu.sync_copy(
        acc.at[pl.ds(sid * _RPS, _RPS)],
        out_hbm.at[cid, pl.ds(sid * _RPS, _RPS)],
    )


def _combine_body(p_ref, o_ref):
    o_ref[...] = p_ref[0, :_N] + p_ref[1, :_N]


_combine = pl.pallas_call(
    _combine_body,
    out_shape=jax.ShapeDtypeStruct((_N, _D), jnp.float32),
)


@jax.jit
def kernel(adj_indices, adj_values, embeds):
    adj = adj_indices.astype(jnp.int32)
    partials = _spmm_sc(adj[1], adj[0], adj_values, embeds)
    return _combine(partials)


# speedup vs baseline: 1.0936x; 1.0799x over previous
import functools

import jax
import jax.numpy as jnp
from jax import lax
from jax.experimental import pallas as pl
from jax.experimental.pallas import tpu as pltpu
from jax.experimental.pallas import tpu_sc as plsc

_N = 10000
_E = 320000
_D = 128
_NC = 2
_NS = 16
_NW = _NC * _NS
_EPW = _E // _NW
_CHUNK = 80
_NCHUNK = _EPW // _CHUNK
_NBUF = 4
_MAIN = 124
_NP = 10240
_RPS = _NP // _NS
_ZROWS = 32

_mesh = plsc.VectorSubcoreMesh(
    core_axis_name="c", subcore_axis_name="s", num_cores=_NC, num_subcores=_NS
)


@functools.partial(
    pl.kernel,
    out_type=jax.ShapeDtypeStruct((_NC, _NP, _D), jnp.float32),
    mesh=_mesh,
    scratch_types=(
        [
            pltpu.VMEM((_ZROWS, _D), jnp.float32),
            pltpu.VMEM_SHARED((_NP, _D), jnp.float32),
        ]
        + [pltpu.VMEM((_CHUNK,), jnp.int32)] * _NBUF
        + [pltpu.VMEM((_CHUNK,), jnp.int32)] * _NBUF
        + [pltpu.VMEM((_CHUNK,), jnp.float32)] * _NBUF
        + [pltpu.VMEM((_CHUNK, _D), jnp.float32)] * _NBUF
        + [pltpu.SemaphoreType.DMA] * (3 * _NBUF)
    ),
)
def _spmm_sc(col_hbm, row_hbm, val_hbm, emb_hbm, out_hbm, zbuf, acc, *bufs_sems):
    colb = bufs_sems[:_NBUF]
    rowb = bufs_sems[_NBUF:2 * _NBUF]
    valb = bufs_sems[2 * _NBUF:3 * _NBUF]
    rbuf = bufs_sems[3 * _NBUF:4 * _NBUF]
    isem = bufs_sems[4 * _NBUF:5 * _NBUF]
    gsem = bufs_sems[5 * _NBUF:6 * _NBUF]
    ssem = bufs_sems[6 * _NBUF:]
    cid = lax.axis_index("c")
    sid = lax.axis_index("s")
    wid = sid * _NC + cid

    def zero_body(i, carry):
        for j in range(_D // 16):
            zbuf[i, pl.ds(j * 16, 16)] = jnp.zeros((16,), jnp.float32)
        return carry

    lax.fori_loop(0, _ZROWS, zero_body, 0)
    for t in range(_RPS // _ZROWS):
        pltpu.sync_copy(zbuf, acc.at[pl.ds(sid * _RPS + t * _ZROWS, _ZROWS)])
    plsc.subcore_barrier()

    def stage_rec(i, b):
        base = wid * _EPW + i * _CHUNK
        pltpu.async_copy(col_hbm.at[pl.ds(base, _CHUNK)], colb[b], isem[b])
        pltpu.async_copy(row_hbm.at[pl.ds(base, _CHUNK)], rowb[b], isem[b])
        pltpu.async_copy(val_hbm.at[pl.ds(base, _CHUNK)], valb[b], isem[b])

    def wait_rec(b):
        pltpu.make_async_copy(col_hbm.at[pl.ds(0, _CHUNK)], colb[b], isem[b]).wait()
        pltpu.make_async_copy(row_hbm.at[pl.ds(0, _CHUNK)], rowb[b], isem[b]).wait()
        pltpu.make_async_copy(val_hbm.at[pl.ds(0, _CHUNK)], valb[b], isem[b]).wait()

    def start_gather(b):
        pltpu.async_copy(emb_hbm.at[colb[b]], rbuf[b], gsem[b])

    def wait_gather(b):
        pltpu.make_async_copy(emb_hbm.at[colb[b]], rbuf[b], gsem[b]).wait()

    def start_scatter(b):
        pltpu.async_copy(rbuf[b], acc.at[rowb[b]], ssem[b], add=True)

    def wait_scatter(b):
        pltpu.make_async_copy(rbuf[b], acc.at[rowb[b]], ssem[b]).wait()

    def mul_rows(b):
        def mul_body(g, c2):
            vblk = valb[b][pl.ds(g * 16, 16)]
            for e16 in range(16):
                s = vblk[e16]
                e = g * 16 + e16
                for j in range(_D // 16):
                    sl = pl.ds(j * 16, 16)
                    rbuf[b][e, sl] = rbuf[b][e, sl] * s
            return c2

        lax.fori_loop(0, _CHUNK // 16, mul_body, 0)

    stage_rec(0, 0)
    stage_rec(1, 1)
    stage_rec(2, 2)
    wait_rec(0)
    start_gather(0)
    wait_rec(1)
    start_gather(1)

    def super_body(k, carry):
        for b in range(_NBUF):
            i = k + b
            bp = (b + _NBUF - 1) % _NBUF
            b2 = (b + 2) % _NBUF

            @pl.when(i >= 1)
            def _wait_prev_scatter():
                wait_scatter(bp)

            @pl.when(i + 3 < _NCHUNK)
            def _stage():
                stage_rec(i + 3, bp)

            @pl.when(i + 2 < _NCHUNK)
            def _prefetch():
                wait_rec(b2)
                start_gather(b2)

            wait_gather(b)
            mul_rows(b)
            start_scatter(b)
        return carry

    lax.fori_loop(0, _MAIN // _NBUF, lambda k, c: super_body(k * _NBUF, c), 0)

    wait_scatter(3)
    wait_gather(0)
    mul_rows(0)
    start_scatter(0)
    wait_scatter(0)

    plsc.subcore_barrier()

    pltpu.sync_copy(
        acc.at[pl.ds(sid * _RPS, _RPS)],
        out_hbm.at[cid, pl.ds(sid * _RPS, _RPS)],
    )


def _combine_body(p_ref, o_ref):
    o_ref[...] = p_ref[0, :_N] + p_ref[1, :_N]


_combine = pl.pallas_call(
    _combine_body,
    out_shape=jax.ShapeDtypeStruct((_N, _D), jnp.float32),
)


@jax.jit
def kernel(adj_indices, adj_values, embeds):
    adj = adj_indices.astype(jnp.int32)
    partials = _spmm_sc(adj[1], adj[0], adj_values, embeds)
    return _combine(partials)
